# baseline (speedup 1.0000x reference)
import jax
import jax.numpy as jnp
from jax.experimental import pallas as pl
from jax.experimental.pallas import tpu as pltpu

B, N, T, P, D, O = 16, 2048, 1, 6, 128, 128
ROWS = 512


def _fused_body(embs_ref, vis_ref, wt_ref, b_ref, mask_ref, out_ref):
    combined = embs_ref[0, :, 0, 0, :]
    scale = jnp.ones((ROWS, 1), jnp.float32)
    for i in range(1, P):
        v = vis_ref[0, :, 0, i : i + 1]
        combined = combined + embs_ref[0, :, 0, i, :] * v
        scale = scale + v
    lin = jnp.dot(combined, wt_ref[:, :], preferred_element_type=jnp.float32)
    res = lin + scale * b_ref[0, :]
    mask = mask_ref[0, :, :]
    out_ref[0, :, :] = jnp.where(mask, res, 0.0)


@jax.jit
def kernel(embs, vis, W, b, masks):
    wt = W.T
    b2 = b.reshape(1, O)
    grid = (B, N // ROWS)
    out = pl.pallas_call(
        _fused_body,
        grid=grid,
        in_specs=[
            pl.BlockSpec((1, ROWS, 1, P, D), lambda i, j: (i, j, 0, 0, 0)),
            pl.BlockSpec((1, ROWS, 1, P), lambda i, j: (i, j, 0, 0)),
            pl.BlockSpec((D, O), lambda i, j: (0, 0)),
            pl.BlockSpec((1, O), lambda i, j: (0, 0)),
            pl.BlockSpec((1, ROWS, 1), lambda i, j: (i, j, 0)),
        ],
        out_specs=pl.BlockSpec((1, ROWS, O), lambda i, j: (i, j, 0)),
        out_shape=jax.ShapeDtypeStruct((B, N, O), jnp.float32),
        compiler_params=pltpu.CompilerParams(
            dimension_semantics=("arbitrary", "arbitrary"),
        ),
    )(embs, vis, wt, b2, masks)
    return out


# speedup vs baseline: 4.8674x; 4.8674x over previous
import jax
import jax.numpy as jnp
from jax import lax
from jax.experimental import pallas as pl
from jax.experimental.pallas import tpu as pltpu

B, N, T, P, D, O = 16, 2048, 1, 6, 128, 128
RN = 512
RE = RN * P
NB = N // RN
CH = 64
CW = CH * P
NCH = RN // CH
WR = RE // 128


def _fused_body(embs_ref, w_ref, wt_ref, b_ref, mask_ref, out_ref):
    ig = lax.broadcasted_iota(jnp.int32, (CH, CW), 1)
    inn = lax.broadcasted_iota(jnp.int32, (CH, CW), 0)
    seg_lo = inn * P
    band = (ig >= seg_lo) & (ig < seg_lo + P)
    is_p0 = ig == seg_lo
    w_all = w_ref[:, :]
    e = embs_ref[:, :]
    cs = []
    scs = []
    for c in range(NCH):
        w_c = jnp.concatenate(
            [w_all[3 * c + k : 3 * c + k + 1, :] for k in range(3)], axis=1
        )
        wb = jnp.broadcast_to(w_c, (CH, CW))
        wb = jnp.where(is_p0, 1.0, wb)
        svis = jnp.where(band, wb, 0.0)
        e_c = e[c * CW : (c + 1) * CW, :]
        cs.append(jnp.dot(svis, e_c, preferred_element_type=jnp.float32))
        scs.append(jnp.sum(svis, axis=1, keepdims=True))
    combined = jnp.concatenate(cs, axis=0)
    scale = jnp.concatenate(scs, axis=0)
    lin = jnp.dot(combined, wt_ref[:, :], preferred_element_type=jnp.float32)
    res = lin + scale * b_ref[0, :]
    out_ref[0, :, :] = jnp.where(mask_ref[0, :, :], res, 0.0)


@jax.jit
def kernel(embs, vis, W, b, masks):
    e2 = embs.reshape(B * N * P, D)
    wflat = vis.reshape(B * N * P // 128, 128)
    wt = W.T
    b2 = b.reshape(1, O)
    grid = (B, NB)
    out = pl.pallas_call(
        _fused_body,
        grid=grid,
        in_specs=[
            pl.BlockSpec((RE, D), lambda i, j: (i * NB + j, 0)),
            pl.BlockSpec((WR, 128), lambda i, j: (i * NB + j, 0)),
            pl.BlockSpec((D, O), lambda i, j: (0, 0)),
            pl.BlockSpec((1, O), lambda i, j: (0, 0)),
            pl.BlockSpec((1, RN, 1), lambda i, j: (i, j, 0)),
        ],
        out_specs=pl.BlockSpec((1, RN, O), lambda i, j: (i, j, 0)),
        out_shape=jax.ShapeDtypeStruct((B, N, O), jnp.float32),
        compiler_params=pltpu.CompilerParams(
            dimension_semantics=("arbitrary", "arbitrary"),
        ),
    )(e2, wflat, wt, b2, masks)
    return out
